# baseline (speedup 1.0000x reference)
import functools

import jax
import jax.numpy as jnp
from jax import lax
from jax.experimental import pallas as pl
from jax.experimental.pallas import tpu as pltpu
from jax.experimental.pallas import tpu_sc as plsc

_MIN_SPEED = 0.5
_MAX_SPEED = 2.0

_NUM_CORES = 2
_NUM_SUBCORES = 16
_NW = _NUM_CORES * _NUM_SUBCORES
_L = 16


def _make_sc_call(N, C, T, ROW):
    UL = T * ROW
    ULP = ((UL + 15) // 16) * 16
    NG = (T + _L - 1) // _L
    TP = ((NG * _L + 15) // 16) * 16
    LWS = 2 * TP
    OV = NG * _L * ROW
    OVP = max(OV, ULP)
    UNITS = N * C
    UPW = UNITS // _NW
    assert UNITS % _NW == 0 and UPW % 2 == 0
    assert ULP % 16 == 0 and LWS % 16 == 0

    mesh = plsc.VectorSubcoreMesh(
        core_axis_name="c", subcore_axis_name="s",
        num_cores=_NUM_CORES, num_subcores=_NUM_SUBCORES)

    def compute(in_v, lw_v, out_v):
        iota = lax.iota(jnp.int32, _L)

        def group_body(g, gcarry):
            lf = lw_v[pl.ds(g * _L, _L)]
            w = lw_v[pl.ds(TP + g * _L, _L)]
            li = lf.astype(jnp.int32)
            ri = jnp.minimum(li + 1, T - 1)
            bl = li * ROW
            br = ri * ROW
            ob = (g * _L + iota) * ROW
            for j in range(ROW):
                a = plsc.load_gather(in_v, [bl])
                b = plsc.load_gather(in_v, [br])
                res = a + w * (b - a)
                plsc.store_scatter(out_v, [ob], res)
                if j + 1 < ROW:
                    bl = bl + 1
                    br = br + 1
                    ob = ob + 1
            return gcarry

        lax.fori_loop(0, NG, group_body, 0)

    @functools.partial(
        pl.kernel,
        out_type=jax.ShapeDtypeStruct((UNITS * ULP,), jnp.float32),
        mesh=mesh,
        scratch_types=[
            pltpu.VMEM((ULP,), jnp.float32),
            pltpu.VMEM((ULP,), jnp.float32),
            pltpu.VMEM((OVP,), jnp.float32),
            pltpu.VMEM((OVP,), jnp.float32),
            pltpu.VMEM((LWS,), jnp.float32),
            pltpu.VMEM((LWS,), jnp.float32),
            pltpu.SemaphoreType.DMA,
            pltpu.SemaphoreType.DMA,
            pltpu.SemaphoreType.DMA,
            pltpu.SemaphoreType.DMA,
        ],
        compiler_params=pltpu.CompilerParams(needs_layout_passes=False),
    )
    def sc_call(x_hbm, lw_hbm, out_hbm,
                in0, in1, o0, o1, lw0, lw1, si0, si1, so0, so1):
        wid = lax.axis_index("s") * _NUM_CORES + lax.axis_index("c")
        base = wid * UPW
        ins, outs, lws = (in0, in1), (o0, o1), (lw0, lw1)
        isems, osems = (si0, si1), (so0, so1)

        def start_in(b, u):
            n = u // C
            pltpu.async_copy(lw_hbm.at[pl.ds(n * LWS, LWS)], lws[b], isems[b])
            pltpu.async_copy(x_hbm.at[pl.ds(u * ULP, ULP)], ins[b], isems[b])

        def wait_in(b):
            pltpu.make_async_copy(
                lw_hbm.at[pl.ds(0, LWS)], lws[b], isems[b]).wait()
            pltpu.make_async_copy(
                x_hbm.at[pl.ds(0, ULP)], ins[b], isems[b]).wait()

        def start_out(b, u):
            pltpu.async_copy(outs[b].at[pl.ds(0, ULP)],
                             out_hbm.at[pl.ds(u * ULP, ULP)], osems[b])

        def wait_out(b):
            pltpu.make_async_copy(outs[b].at[pl.ds(0, ULP)],
                                  out_hbm.at[pl.ds(0, ULP)], osems[b]).wait()

        start_in(0, base)
        start_in(1, base + 1)

        def k2_body(k2, carry):
            for b in (0, 1):
                u = base + k2 * 2 + b
                wait_in(b)

                @pl.when(k2 > 0)
                def _():
                    wait_out(b)

                compute(ins[b], lws[b], outs[b])
                start_out(b, u)

                @pl.when(k2 < UPW // 2 - 1)
                def _():
                    start_in(b, u + 2)
            return carry

        lax.fori_loop(0, UPW // 2, k2_body, 0)
        wait_out(0)
        wait_out(1)

    return sc_call, ULP, TP, LWS


def kernel(x):
    N, C, T, V, M = x.shape
    ROW = V * M
    UL = T * ROW

    skey = jax.random.key(42)
    speed = (jax.random.uniform(skey, (N,), dtype=jnp.float32)
             * (_MAX_SPEED - _MIN_SPEED) + _MIN_SPEED)

    sc_call, ULP, TP, LWS = _make_sc_call(N, C, T, ROW)

    t = jnp.arange(T, dtype=jnp.float32)[None, :]
    t_new = jnp.clip(t / speed[:, None], 0.0, float(T - 1))
    lf = jnp.floor(t_new)
    w = t_new - lf
    lw = jnp.zeros((N, LWS), jnp.float32)
    lw = lw.at[:, :T].set(lf).at[:, TP:TP + T].set(w)

    x2 = jnp.pad(x.reshape(N * C, UL), ((0, 0), (0, ULP - UL)))
    out_flat = sc_call(x2.reshape(-1), lw.reshape(-1))
    out = out_flat.reshape(N * C, ULP)[:, :UL]
    return out.reshape(N, C, T, V, M), speed
